# baseline (speedup 1.0000x reference)
import jax
import jax.numpy as jnp
from jax.experimental import pallas as pl
from jax.experimental.pallas import tpu as pltpu

_CC = 200


def _loss_step(lab_ref, x_ref, out_ref):
    x = x_ref[...]
    T = x.shape[1]
    s = jnp.sum(x, axis=1)
    d = s - 1.0
    part = jnp.sum(d * d)
    lab = lab_ref[0, :]
    c_idx = (jax.lax.broadcasted_iota(jnp.int32, s.shape, 0)
             + pl.program_id(0) * _CC)
    corr = jnp.sum(jnp.where(lab[None, :] == c_idx, 99.0 - 18.0 * s, 0.0))
    out_ref[...] = ((part + corr) * (0.5 / T)).reshape(1, 1, 1)


def kernel(outputs, labels):
    B, C, H, W, T = outputs.shape
    xt = jnp.transpose(outputs.reshape(B, C, T), (1, 2, 0))
    n_steps = C // _CC
    lab2 = labels.reshape(1, B)
    out = pl.pallas_call(
        _loss_step,
        grid=(n_steps,),
        in_specs=[
            pl.BlockSpec((1, B), lambda i: (0, 0)),
            pl.BlockSpec((_CC, T, B), lambda i: (i, 0, 0)),
        ],
        out_specs=pl.BlockSpec((1, 1, 1), lambda i: (i, 0, 0)),
        out_shape=jax.ShapeDtypeStruct((n_steps, 1, 1), jnp.float32),
        compiler_params=pltpu.CompilerParams(
            dimension_semantics=("parallel",)),
    )(lab2, xt)
    return jnp.sum(out)


# speedup vs baseline: 1.0540x; 1.0540x over previous
import jax
import jax.numpy as jnp
from jax.experimental import pallas as pl
from jax.experimental.pallas import tpu as pltpu

_CC = 125


def _loss_step(lab_ref, x_ref, out_ref):
    x = x_ref[...]
    T = x.shape[1]
    s = jnp.sum(x, axis=1)
    d = s - 1.0
    part = jnp.sum(d * d)
    lab = lab_ref[0, :]
    c_idx = (jax.lax.broadcasted_iota(jnp.int32, s.shape, 0)
             + pl.program_id(0) * _CC)
    corr = jnp.sum(jnp.where(lab[None, :] == c_idx, 99.0 - 18.0 * s, 0.0))
    out_ref[...] = ((part + corr) * (0.5 / T)).reshape(1, 1, 1)


def kernel(outputs, labels):
    B, C, H, W, T = outputs.shape
    xt = jnp.transpose(outputs.reshape(B, C, T), (1, 2, 0))
    n_steps = C // _CC
    lab2 = labels.reshape(1, B)
    out = pl.pallas_call(
        _loss_step,
        grid=(n_steps,),
        in_specs=[
            pl.BlockSpec((1, B), lambda i: (0, 0)),
            pl.BlockSpec((_CC, T, B), lambda i: (i, 0, 0)),
        ],
        out_specs=pl.BlockSpec((1, 1, 1), lambda i: (i, 0, 0)),
        out_shape=jax.ShapeDtypeStruct((n_steps, 1, 1), jnp.float32),
        compiler_params=pltpu.CompilerParams(
            dimension_semantics=("parallel",)),
    )(lab2, xt)
    return jnp.sum(out)
